# Initial kernel scaffold; baseline (speedup 1.0000x reference)
import jax
import jax.numpy as jnp
from jax.experimental import pallas as pl


def kernel(k_new, v_new, cache_seqlens, qcache_seqlens, k_cache_buf, v_cache_buf):
    raise NotImplementedError("write your pallas kernel here")


# speedup vs baseline: 6.6788x; 6.6788x over previous
import jax
import jax.numpy as jnp
from jax.experimental import pallas as pl
from jax.experimental.pallas import tpu as pltpu

B, H, S, D, U = 8, 16, 2048, 128, 32


def _body(pos_ref, k_new_ref, v_new_ref, k_cache_ref, v_cache_ref, out_ref):
    b = pl.program_id(0)
    pos = pos_ref[b]
    out_ref[0, 0, 0] = k_cache_ref[0, 0]
    out_ref[1, 0, 0] = v_cache_ref[0, 0]
    out_ref[0, 0, 0, pl.ds(pos, U), :] = k_new_ref[0, 0]
    out_ref[1, 0, 0, pl.ds(pos, U), :] = v_new_ref[0, 0]


def kernel(k_new, v_new, cache_seqlens, qcache_seqlens, k_cache_buf, v_cache_buf):
    pos = (cache_seqlens - qcache_seqlens).astype(jnp.int32)

    grid_spec = pltpu.PrefetchScalarGridSpec(
        num_scalar_prefetch=1,
        grid=(B, H),
        in_specs=[
            pl.BlockSpec((1, 1, U, D), lambda b, h, pos_ref: (b, h, 0, 0)),
            pl.BlockSpec((1, 1, U, D), lambda b, h, pos_ref: (b, h, 0, 0)),
            pl.BlockSpec((1, 1, S, D), lambda b, h, pos_ref: (b, h, 0, 0)),
            pl.BlockSpec((1, 1, S, D), lambda b, h, pos_ref: (b, h, 0, 0)),
        ],
        out_specs=pl.BlockSpec(
            (2, 1, 1, S, D), lambda b, h, pos_ref: (0, b, h, 0, 0)
        ),
    )

    return pl.pallas_call(
        _body,
        grid_spec=grid_spec,
        out_shape=jax.ShapeDtypeStruct((2, B, H, S, D), jnp.float32),
    )(pos, k_new, v_new, k_cache_buf, v_cache_buf)
